# baseline (speedup 1.0000x reference)
import functools

import jax
import jax.numpy as jnp
from jax import lax
from jax.experimental import pallas as pl
from jax.experimental.pallas import tpu as pltpu
from jax.experimental.pallas import tpu_sc as plsc

B, V, D, P = 1024, 100000, 16, 20

_NC, _NS = 2, 16
_NW = _NC * _NS
_BP = B * P
_BPW = _BP // _NW


def _sc_gather(table, idx):
    mesh = plsc.VectorSubcoreMesh(core_axis_name="c", subcore_axis_name="s")

    @functools.partial(
        pl.kernel,
        out_type=jax.ShapeDtypeStruct((_BP, D), jnp.float32),
        mesh=mesh,
        scratch_types=[
            pltpu.VMEM((_BPW,), jnp.int32),
            pltpu.VMEM((_BPW, D), jnp.float32),
            pltpu.SemaphoreType.DMA,
        ],
        compiler_params=pltpu.CompilerParams(use_tc_tiling_on_sc=False),
    )
    def k(table_hbm, idx_hbm, out_hbm, idx_v, rows_v, sem):
        wid = lax.axis_index("s") * _NC + lax.axis_index("c")
        base = wid * _BPW
        pltpu.sync_copy(idx_hbm.at[pl.ds(base, _BPW)], idx_v)
        pltpu.async_copy(table_hbm.at[idx_v], rows_v, sem).wait()
        pltpu.sync_copy(rows_v, out_hbm.at[pl.ds(base, _BPW)])

    return k(table, idx)


_VT = 2048
_NBLK = (V + _VT - 1) // _VT
_VLAST = V - (_NBLK - 1) * _VT


def _tc_body(iv_ref, w_ref, pvt_ref, hc_ref, out_ref, acc_ref):
    i = pl.program_id(0)

    @pl.when(i == 0)
    def _init():
        acc_ref[...] = jnp.zeros_like(acc_ref)

    def contrib(a, b):
        return lax.dot_general(a, b, (((1,), (1,)), ((), ())),
                               preferred_element_type=jnp.float32)

    @pl.when(i < _NBLK - 1)
    def _full():
        acc_ref[...] += contrib(iv_ref[...], w_ref[...])

    @pl.when(i == _NBLK - 1)
    def _last():
        m = lax.broadcasted_iota(jnp.int32, (1, _VT), 1) < _VLAST
        iv = jnp.where(m, iv_ref[...], 0.0)
        w = jnp.where(m, w_ref[...], 0.0)
        acc_ref[...] += contrib(iv, w)

        x = acc_ref[...]
        logits = jnp.zeros((B, P), jnp.float32)
        for d in range(D):
            logits = logits + pvt_ref[d] * x[:, d:d + 1]
        t = hc_ref[...].astype(jnp.float32)
        bce = (jnp.maximum(logits, 0.0) - logits * t
               + jnp.log1p(jnp.exp(-jnp.abs(logits))))
        out_ref[0, 0] = jnp.sum(bce) * (1.0 / (B * P))


def _tc_loss(inputs_vector, W, pvt, hc):
    out = pl.pallas_call(
        _tc_body,
        grid=(_NBLK,),
        in_specs=[
            pl.BlockSpec((B, _VT), lambda i: (0, i)),
            pl.BlockSpec((D, _VT), lambda i: (0, i)),
            pl.BlockSpec((D, B, P), lambda i: (0, 0, 0)),
            pl.BlockSpec((B, P), lambda i: (0, 0)),
        ],
        out_specs=pl.BlockSpec(memory_space=pltpu.SMEM),
        out_shape=jax.ShapeDtypeStruct((1, 1), jnp.float32),
        scratch_shapes=[pltpu.VMEM((B, D), jnp.float32)],
    )(inputs_vector, W, pvt, hc)
    return out


def kernel(inputs_vector, path_nodes_indices, huffman_codes, W, cls):
    idx = path_nodes_indices.astype(jnp.int32).reshape(_BP)
    rows = _sc_gather(cls, idx)
    pvt = rows.reshape(B, P, D).transpose(2, 0, 1)
    loss = _tc_loss(inputs_vector, W, pvt,
                    huffman_codes.astype(jnp.int32))
    return loss.reshape(1)


# speedup vs baseline: 1.0151x; 1.0151x over previous
import functools

import jax
import jax.numpy as jnp
from jax import lax
from jax.experimental import pallas as pl
from jax.experimental.pallas import tpu as pltpu
from jax.experimental.pallas import tpu_sc as plsc

B, V, D, P = 1024, 100000, 16, 20

_NC, _NS = 2, 16
_NW = _NC * _NS
_BP = B * P
_BPW = _BP // _NW


def _sc_gather(table, idx):
    mesh = plsc.VectorSubcoreMesh(core_axis_name="c", subcore_axis_name="s")

    @functools.partial(
        pl.kernel,
        out_type=jax.ShapeDtypeStruct((_BP, D), jnp.float32),
        mesh=mesh,
        scratch_types=[
            pltpu.VMEM((_BPW,), jnp.int32),
            pltpu.VMEM((_BPW, D), jnp.float32),
            pltpu.SemaphoreType.DMA,
        ],
        compiler_params=pltpu.CompilerParams(use_tc_tiling_on_sc=False),
    )
    def k(table_hbm, idx_hbm, out_hbm, idx_v, rows_v, sem):
        wid = lax.axis_index("s") * _NC + lax.axis_index("c")
        base = wid * _BPW
        pltpu.sync_copy(idx_hbm.at[pl.ds(base, _BPW)], idx_v)
        pltpu.async_copy(table_hbm.at[idx_v], rows_v, sem).wait()
        pltpu.sync_copy(rows_v, out_hbm.at[pl.ds(base, _BPW)])

    return k(table, idx)


_VT = 2048
_NBLK = (V + _VT - 1) // _VT
_VLAST = V - (_NBLK - 1) * _VT


def _tc_body(iv_ref, w_ref, pvt_ref, hc_ref, out_ref, acc_ref):
    i = pl.program_id(0)

    @pl.when(i == 0)
    def _init():
        acc_ref[...] = jnp.zeros_like(acc_ref)

    def contrib(a, b):
        return lax.dot_general(a.astype(jnp.bfloat16), b.astype(jnp.bfloat16),
                               (((1,), (1,)), ((), ())),
                               preferred_element_type=jnp.float32)

    @pl.when(i < _NBLK - 1)
    def _full():
        acc_ref[...] += contrib(iv_ref[...], w_ref[...])

    @pl.when(i == _NBLK - 1)
    def _last():
        m = lax.broadcasted_iota(jnp.int32, (1, _VT), 1) < _VLAST
        iv = jnp.where(m, iv_ref[...], 0.0)
        w = jnp.where(m, w_ref[...], 0.0)
        acc_ref[...] += contrib(iv, w)

        x = acc_ref[...]
        logits = jnp.zeros((B, P), jnp.float32)
        for d in range(D):
            logits = logits + pvt_ref[d] * x[:, d:d + 1]
        t = hc_ref[...].astype(jnp.float32)
        bce = (jnp.maximum(logits, 0.0) - logits * t
               + jnp.log1p(jnp.exp(-jnp.abs(logits))))
        out_ref[0, 0] = jnp.sum(bce) * (1.0 / (B * P))


def _tc_loss(inputs_vector, W, pvt, hc):
    out = pl.pallas_call(
        _tc_body,
        grid=(_NBLK,),
        in_specs=[
            pl.BlockSpec((B, _VT), lambda i: (0, i)),
            pl.BlockSpec((D, _VT), lambda i: (0, i)),
            pl.BlockSpec((D, B, P), lambda i: (0, 0, 0)),
            pl.BlockSpec((B, P), lambda i: (0, 0)),
        ],
        out_specs=pl.BlockSpec(memory_space=pltpu.SMEM),
        out_shape=jax.ShapeDtypeStruct((1, 1), jnp.float32),
        scratch_shapes=[pltpu.VMEM((B, D), jnp.float32)],
    )(inputs_vector, W, pvt, hc)
    return out


def kernel(inputs_vector, path_nodes_indices, huffman_codes, W, cls):
    idx = path_nodes_indices.astype(jnp.int32).reshape(_BP)
    rows = _sc_gather(cls, idx)
    pvt = rows.reshape(B, P, D).transpose(2, 0, 1)
    loss = _tc_loss(inputs_vector, W, pvt,
                    huffman_codes.astype(jnp.int32))
    return loss.reshape(1)
